# baseline (speedup 1.0000x reference)
import jax
import jax.numpy as jnp
from jax import lax
from jax.experimental import pallas as pl
from jax.experimental.pallas import tpu as pltpu
from jax.experimental.pallas import tpu_sc as plsc

VOCAB_SIZE = 100000
N_EMBED = 768
CONTEXT_SIZE = 2048
BATCH = 4
SEQ_LEN = 2048

NUM_CORES = 2
NUM_SUBCORES = 16
NUM_WORKERS = NUM_CORES * NUM_SUBCORES
S_PER_W = SEQ_LEN // NUM_WORKERS
LANES = 16
CHUNKS = N_EMBED // LANES
HALF = S_PER_W // 2
N_CHUNK = BATCH * 2


def _body(ids_hbm, wte_hbm, wpe_hbm, out_hbm,
          idx_v, wpe_v, rows0, rows1, sg0, sg1, so0, so1):
    wid = lax.axis_index("s") * NUM_CORES + lax.axis_index("c")
    s0 = wid * S_PER_W

    bufs = (rows0, rows1)
    gsems = (sg0, sg1)
    osems = (so0, so1)

    for b in range(BATCH):
        pltpu.sync_copy(ids_hbm.at[b, pl.ds(s0, S_PER_W)], idx_v.at[b])
    pltpu.sync_copy(wpe_hbm.at[pl.ds(s0, S_PER_W)], wpe_v)

    def gather(c):
        b, h = divmod(c, 2)
        return pltpu.async_copy(
            wte_hbm.at[idx_v.at[b, pl.ds(h * HALF, HALF)]],
            bufs[c % 2], gsems[c % 2])

    def add_wpe(c):
        buf = bufs[c % 2]
        h = (c % 2) * HALF

        def add_row(i, _):
            for j in range(CHUNKS):
                x = wpe_v[h + i, pl.ds(j * LANES, LANES)]
                plsc.addupdate(buf.at[i, pl.ds(j * LANES, LANES)], x)
            return 0

        lax.fori_loop(0, HALF, add_row, 0)

    def writeback(c):
        b, h = divmod(c, 2)
        return pltpu.async_copy(
            bufs[c % 2], out_hbm.at[b, pl.ds(s0 + h * HALF, HALF)],
            osems[c % 2])

    gd = gather(0)
    wr = [None, None]
    for c in range(N_CHUNK):
        gd.wait()
        if c + 1 < N_CHUNK:
            if wr[(c + 1) % 2] is not None:
                wr[(c + 1) % 2].wait()
            gd = gather(c + 1)
        add_wpe(c)
        wr[c % 2] = writeback(c)
    wr[0].wait()
    wr[1].wait()


@jax.jit
def _embed(input_ids, wte, wpe):
    mesh = plsc.VectorSubcoreMesh(core_axis_name="c", subcore_axis_name="s")
    return pl.kernel(
        _body,
        out_type=jax.ShapeDtypeStruct((BATCH, SEQ_LEN, N_EMBED), jnp.float32),
        mesh=mesh,
        scratch_types=[
            pltpu.VMEM((BATCH, S_PER_W), jnp.int32),
            pltpu.VMEM((S_PER_W, N_EMBED), jnp.float32),
            pltpu.VMEM((HALF, N_EMBED), jnp.float32),
            pltpu.VMEM((HALF, N_EMBED), jnp.float32),
            pltpu.SemaphoreType.DMA,
            pltpu.SemaphoreType.DMA,
            pltpu.SemaphoreType.DMA,
            pltpu.SemaphoreType.DMA,
        ],
    )(input_ids, wte, wpe)


def kernel(input_ids, wte, wpe):
    return _embed(input_ids.astype(jnp.int32), wte, wpe)


# speedup vs baseline: 1.1896x; 1.1896x over previous
import jax
import jax.numpy as jnp
from jax import lax
from jax.experimental import pallas as pl
from jax.experimental.pallas import tpu as pltpu
from jax.experimental.pallas import tpu_sc as plsc

VOCAB_SIZE = 100000
N_EMBED = 768
CONTEXT_SIZE = 2048
BATCH = 4
SEQ_LEN = 2048

NUM_CORES = 2
NUM_SUBCORES = 16
NUM_WORKERS = NUM_CORES * NUM_SUBCORES
S_PER_W = SEQ_LEN // NUM_WORKERS
LANES = 16
CHUNKS = N_EMBED // LANES
NQ = 4
Q = S_PER_W // NQ


def _body(ids_hbm, wte_hbm, wpe_hbm, out_hbm,
          idx_v, wpe_v, r0, r1, r2, r3,
          sg0, sg1, sg2, sg3, so0, so1, so2, so3):
    wid = lax.axis_index("s") * NUM_CORES + lax.axis_index("c")
    s0 = wid * S_PER_W

    bufs = (r0, r1, r2, r3)
    gsems = (sg0, sg1, sg2, sg3)
    osems = (so0, so1, so2, so3)

    for b in range(BATCH):
        pltpu.sync_copy(ids_hbm.at[b, pl.ds(s0, S_PER_W)], idx_v.at[b])
    pltpu.sync_copy(wpe_hbm.at[pl.ds(s0, S_PER_W)], wpe_v)

    def gd(bat, q):
        return pltpu.make_async_copy(
            wte_hbm.at[idx_v.at[bat, pl.ds(q * Q, Q)]], bufs[q], gsems[q])

    def wr(bat, q):
        return pltpu.make_async_copy(
            bufs[q], out_hbm.at[bat, pl.ds(s0 + q * Q, Q)], osems[q])

    def add_wpe(q):
        buf = bufs[q]

        def add_row(i, _):
            for j in range(CHUNKS):
                x = wpe_v[q * Q + i, pl.ds(j * LANES, LANES)]
                plsc.addupdate(buf.at[i, pl.ds(j * LANES, LANES)], x)
            return 0

        lax.fori_loop(0, Q, add_row, 0)

    gd(0, 0).start()
    gd(0, 1).start()

    def batch_body(g, _):
        for q in range(NQ):
            gd(g, q).wait()
            if q < 2:
                @pl.when(g > 0)
                def _w(g=g, q=q):
                    wr(g - 1, q + 2).wait()
                gd(g, q + 2).start()
            else:
                @pl.when(g < BATCH - 1)
                def _p(g=g, q=q):
                    wr(g, q - 2).wait()
                    gd(g + 1, q - 2).start()
            add_wpe(q)
            wr(g, q).start()
        return 0

    lax.fori_loop(0, BATCH, batch_body, 0)
    for q in range(NQ):
        wr(BATCH - 1, q).wait()


@jax.jit
def _embed(input_ids, wte, wpe):
    mesh = plsc.VectorSubcoreMesh(core_axis_name="c", subcore_axis_name="s")
    return pl.kernel(
        _body,
        out_type=jax.ShapeDtypeStruct((BATCH, SEQ_LEN, N_EMBED), jnp.float32),
        mesh=mesh,
        scratch_types=(
            [pltpu.VMEM((BATCH, S_PER_W), jnp.int32),
             pltpu.VMEM((S_PER_W, N_EMBED), jnp.float32)]
            + [pltpu.VMEM((Q, N_EMBED), jnp.float32) for _ in range(NQ)]
            + [pltpu.SemaphoreType.DMA for _ in range(2 * NQ)]
        ),
    )(input_ids, wte, wpe)


def kernel(input_ids, wte, wpe):
    return _embed(input_ids.astype(jnp.int32), wte, wpe)
